# baseline (speedup 1.0000x reference)
import functools

import jax
import jax.numpy as jnp
from jax import lax
from jax.experimental import pallas as pl
from jax.experimental.pallas import tpu as pltpu
from jax.experimental.pallas import tpu_sc as plsc

NC = 2
NS = 16
L = 16
CHUNK = 64


def _lane_sum(x):
    lanes = lax.iota(jnp.int32, L)
    for sh in (1, 2, 4, 8):
        perm = lanes ^ sh
        x = x + x.at[perm].get(mode="promise_in_bounds")
    return x


def _rsqrt_nr(x, iters):
    i = lax.bitcast_convert_type(x, jnp.int32)
    i = 0x5F3759DF - (i >> 1)
    y = lax.bitcast_convert_type(i, jnp.float32)
    xh = x * 0.5
    for _ in range(iters):
        y = y * (1.5 - xh * y * y)
    return y


@functools.lru_cache(maxsize=None)
def _make_sc_kernel(n_rows, dim):
    assert dim == 128 and n_rows % (NC * NS * CHUNK * 2) == 0
    rows_per_tile = n_rows // (NC * NS)
    n_pairs = rows_per_tile // (2 * CHUNK)
    nj = dim // L

    mesh = plsc.VectorSubcoreMesh(
        core_axis_name="c", subcore_axis_name="s",
        num_cores=NC, num_subcores=NS)

    @functools.partial(
        pl.kernel,
        out_type=(
            jax.ShapeDtypeStruct((n_rows, dim), jnp.float32),
            jax.ShapeDtypeStruct((n_rows, dim), jnp.float32),
        ),
        mesh=mesh,
        scratch_types=[
            pltpu.VMEM((rows_per_tile,), jnp.int32),
            pltpu.VMEM((2, CHUNK, dim), jnp.float32),
            pltpu.VMEM((2, CHUNK, dim), jnp.float32),
            pltpu.VMEM((2, CHUNK, dim), jnp.float32),
            pltpu.VMEM((2, CHUNK, dim), jnp.float32),
            pltpu.VMEM((dim,), jnp.float32),
            pltpu.VMEM((dim,), jnp.float32),
            pltpu.SemaphoreType.DMA,
            pltpu.SemaphoreType.DMA,
            pltpu.SemaphoreType.DMA,
            pltpu.SemaphoreType.DMA,
            pltpu.SemaphoreType.DMA,
        ],
    )
    def sc_kernel(x_hbm, er_hbm, ei_hbm, g_hbm, b_hbm, outr_hbm, outi_hbm,
                  idx_v, re_v, im_v, or_v, oi_v, g_v, b_v,
                  sem_r0, sem_r1, sem_i0, sem_i1, sem_o):
        wid = lax.axis_index("s") * NC + lax.axis_index("c")
        tile_base = wid * rows_per_tile

        pltpu.sync_copy(g_hbm, g_v)
        pltpu.sync_copy(b_hbm, b_v)
        pltpu.sync_copy(x_hbm.at[pl.ds(tile_base, rows_per_tile)], idx_v)
        gs = [g_v[pl.ds(L * j, L)] for j in range(nj)]
        bs = [b_v[pl.ds(L * j, L)] for j in range(nj)]

        def gather_in(k, slot, semr, semi):
            idx = idx_v.at[pl.ds(k * CHUNK, CHUNK)]
            cr = pltpu.async_copy(er_hbm.at[idx], re_v.at[slot], semr)
            ci = pltpu.async_copy(ei_hbm.at[idx], im_v.at[slot], semi)
            return cr, ci

        def _one_row(slot, r):
            mags, ys = [], []
            acc = jnp.zeros((L,), jnp.float32)
            acc2 = jnp.zeros((L,), jnp.float32)
            for j in range(nj):
                re = re_v[slot, r, pl.ds(L * j, L)]
                im = im_v[slot, r, pl.ds(L * j, L)]
                msq = re * re + im * im
                y = _rsqrt_nr(msq, 1)
                mag = msq * y
                acc = acc + mag
                acc2 = acc2 + msq
                mags.append(mag)
                ys.append(y)
            mu_v = _lane_sum(acc) * (1.0 / dim)
            ex2 = _lane_sum(acc2) * (1.0 / dim)
            var = ex2 - mu_v * mu_v
            return mags, ys, mu_v, var

        def _finish_row(slot, r, mags, ys, mu_v, var):
            rs = _rsqrt_nr(var + 1e-5, 2)
            for j in range(nj):
                re = re_v[slot, r, pl.ds(L * j, L)]
                im = im_v[slot, r, pl.ds(L * j, L)]
                normed = (mags[j] - mu_v) * (rs * gs[j]) + bs[j]
                s = normed * ys[j]
                or_v[slot, r, pl.ds(L * j, L)] = re * s
                oi_v[slot, r, pl.ds(L * j, L)] = im * s

        def make_row_body(slot):
            def row_body(q, _):
                r0 = 2 * q
                r1 = r0 + 1
                st0 = _one_row(slot, r0)
                st1 = _one_row(slot, r1)
                _finish_row(slot, r0, *st0)
                _finish_row(slot, r1, *st1)
                return _
            return row_body

        def compute_and_emit(k, slot, cr, ci):
            cr.wait()
            ci.wait()
            lax.fori_loop(0, CHUNK // 2, make_row_body(slot), None)
            base = tile_base + k * CHUNK
            co_r = pltpu.async_copy(or_v.at[slot],
                                    outr_hbm.at[pl.ds(base, CHUNK)], sem_o)
            co_i = pltpu.async_copy(oi_v.at[slot],
                                    outi_hbm.at[pl.ds(base, CHUNK)], sem_o)
            return co_r, co_i

        c0 = gather_in(0, 0, sem_r0, sem_i0)

        def pair_body(p, _):
            k0 = 2 * p
            c1 = gather_in(k0 + 1, 1, sem_r1, sem_i1)
            w0 = compute_and_emit(k0, 0, *c0)

            @pl.when(p + 1 < n_pairs)
            def _():
                gather_in(k0 + 2, 0, sem_r0, sem_i0)

            w1 = compute_and_emit(k0 + 1, 1, *c1)
            w0[0].wait()
            w0[1].wait()
            w1[0].wait()
            w1[1].wait()
            return _

        lax.fori_loop(0, n_pairs, pair_body, None)

    return sc_kernel


def kernel(x, embed_real, embed_imag, gamma, beta):
    b, h = x.shape
    v, d = embed_real.shape
    n = b * h
    xf = x.reshape(n)
    sc = _make_sc_kernel(n, d)
    outr, outi = sc(xf, embed_real, embed_imag, gamma, beta)
    return lax.complex(outr.reshape(b, h, d), outi.reshape(b, h, d))


# speedup vs baseline: 1.0157x; 1.0157x over previous
import functools

import jax
import jax.numpy as jnp
from jax import lax
from jax.experimental import pallas as pl
from jax.experimental.pallas import tpu as pltpu
from jax.experimental.pallas import tpu_sc as plsc

NC = 2
NS = 16
L = 16
CHUNK = 64


def _lane_sum(x):
    lanes = lax.iota(jnp.int32, L)
    for sh in (1, 2, 4, 8):
        perm = lanes ^ sh
        x = x + x.at[perm].get(mode="promise_in_bounds")
    return x


def _rsqrt_nr(x, iters):
    i = lax.bitcast_convert_type(x, jnp.int32)
    i = 0x5F3759DF - (i >> 1)
    y = lax.bitcast_convert_type(i, jnp.float32)
    xh = x * 0.5
    for _ in range(iters):
        y = y * (1.5 - xh * y * y)
    return y


@functools.lru_cache(maxsize=None)
def _make_sc_kernel(n_rows, dim):
    assert dim == 128 and n_rows % (NC * NS * CHUNK * 2) == 0
    rows_per_tile = n_rows // (NC * NS)
    n_pairs = rows_per_tile // (2 * CHUNK)
    nj = dim // L

    mesh = plsc.VectorSubcoreMesh(
        core_axis_name="c", subcore_axis_name="s",
        num_cores=NC, num_subcores=NS)

    @functools.partial(
        pl.kernel,
        out_type=(
            jax.ShapeDtypeStruct((n_rows, dim), jnp.float32),
            jax.ShapeDtypeStruct((n_rows, dim), jnp.float32),
        ),
        mesh=mesh,
        scratch_types=[
            pltpu.VMEM((rows_per_tile,), jnp.int32),
            pltpu.VMEM((2, CHUNK, dim), jnp.float32),
            pltpu.VMEM((2, CHUNK, dim), jnp.float32),
            pltpu.VMEM((2, CHUNK, dim), jnp.float32),
            pltpu.VMEM((2, CHUNK, dim), jnp.float32),
            pltpu.VMEM((dim,), jnp.float32),
            pltpu.VMEM((dim,), jnp.float32),
            pltpu.SemaphoreType.DMA,
            pltpu.SemaphoreType.DMA,
            pltpu.SemaphoreType.DMA,
            pltpu.SemaphoreType.DMA,
            pltpu.SemaphoreType.DMA,
        ],
    )
    def sc_kernel(x_hbm, er_hbm, ei_hbm, g_hbm, b_hbm, outr_hbm, outi_hbm,
                  idx_v, re_v, im_v, or_v, oi_v, g_v, b_v,
                  sem_r0, sem_r1, sem_i0, sem_i1, sem_o):
        wid = lax.axis_index("s") * NC + lax.axis_index("c")
        tile_base = wid * rows_per_tile

        pltpu.sync_copy(g_hbm, g_v)
        pltpu.sync_copy(b_hbm, b_v)
        pltpu.sync_copy(x_hbm.at[pl.ds(tile_base, rows_per_tile)], idx_v)
        gs = [g_v[pl.ds(L * j, L)] for j in range(nj)]
        bs = [b_v[pl.ds(L * j, L)] for j in range(nj)]

        def gather_in(k, slot, semr, semi):
            idx = idx_v.at[pl.ds(k * CHUNK, CHUNK)]
            cr = pltpu.async_copy(er_hbm.at[idx], re_v.at[slot], semr)
            ci = pltpu.async_copy(ei_hbm.at[idx], im_v.at[slot], semi)
            return cr, ci

        def _one_row(slot, r):
            mags, ys = [], []
            acc = jnp.zeros((L,), jnp.float32)
            acc2 = jnp.zeros((L,), jnp.float32)
            for j in range(nj):
                re = re_v[slot, r, pl.ds(L * j, L)]
                im = im_v[slot, r, pl.ds(L * j, L)]
                msq = re * re + im * im
                y = _rsqrt_nr(msq, 1)
                mag = msq * y
                acc = acc + mag
                acc2 = acc2 + msq
                mags.append(mag)
                ys.append(y)
            mu_v = _lane_sum(acc) * (1.0 / dim)
            ex2 = _lane_sum(acc2) * (1.0 / dim)
            var = ex2 - mu_v * mu_v
            return mags, ys, mu_v, var

        def _finish_row(slot, r, mags, ys, mu_v, var):
            rs = _rsqrt_nr(var + 1e-5, 2)
            for j in range(nj):
                re = re_v[slot, r, pl.ds(L * j, L)]
                im = im_v[slot, r, pl.ds(L * j, L)]
                normed = (mags[j] - mu_v) * (rs * gs[j]) + bs[j]
                s = normed * ys[j]
                or_v[slot, r, pl.ds(L * j, L)] = re * s
                oi_v[slot, r, pl.ds(L * j, L)] = im * s

        def compute_and_emit(k, slot, cr, ci):
            cr.wait()
            ci.wait()

            @plsc.parallel_loop(0, CHUNK, 1, unroll=2)
            def _(r):
                st = _one_row(slot, r)
                _finish_row(slot, r, *st)
            base = tile_base + k * CHUNK
            co_r = pltpu.async_copy(or_v.at[slot],
                                    outr_hbm.at[pl.ds(base, CHUNK)], sem_o)
            co_i = pltpu.async_copy(oi_v.at[slot],
                                    outi_hbm.at[pl.ds(base, CHUNK)], sem_o)
            return co_r, co_i

        c0 = gather_in(0, 0, sem_r0, sem_i0)

        def pair_body(p, _):
            k0 = 2 * p
            c1 = gather_in(k0 + 1, 1, sem_r1, sem_i1)
            w0 = compute_and_emit(k0, 0, *c0)

            @pl.when(p + 1 < n_pairs)
            def _():
                gather_in(k0 + 2, 0, sem_r0, sem_i0)

            w1 = compute_and_emit(k0 + 1, 1, *c1)
            w0[0].wait()
            w0[1].wait()
            w1[0].wait()
            w1[1].wait()
            return _

        lax.fori_loop(0, n_pairs, pair_body, None)

    return sc_kernel


def kernel(x, embed_real, embed_imag, gamma, beta):
    b, h = x.shape
    v, d = embed_real.shape
    n = b * h
    xf = x.reshape(n)
    sc = _make_sc_kernel(n, d)
    outr, outi = sc(xf, embed_real, embed_imag, gamma, beta)
    return lax.complex(outr.reshape(b, h, d), outi.reshape(b, h, d))


# speedup vs baseline: 1.0296x; 1.0136x over previous
import functools

import jax
import jax.numpy as jnp
from jax import lax
from jax.experimental import pallas as pl
from jax.experimental.pallas import tpu as pltpu
from jax.experimental.pallas import tpu_sc as plsc

NC = 2
NS = 16
L = 16
CHUNK = 80


def _lane_sum(x):
    lanes = lax.iota(jnp.int32, L)
    for sh in (1, 2, 4, 8):
        perm = lanes ^ sh
        x = x + x.at[perm].get(mode="promise_in_bounds")
    return x


def _rsqrt_nr(x, iters):
    i = lax.bitcast_convert_type(x, jnp.int32)
    i = 0x5F3759DF - (i >> 1)
    y = lax.bitcast_convert_type(i, jnp.float32)
    xh = x * 0.5
    for _ in range(iters):
        y = y * (1.5 - xh * y * y)
    return y


@functools.lru_cache(maxsize=None)
def _make_sc_kernel(n_rows, dim):
    assert dim == 128 and n_rows % (NC * NS * CHUNK * 2) == 0
    rows_per_tile = n_rows // (NC * NS)
    n_pairs = rows_per_tile // (2 * CHUNK)
    nj = dim // L

    mesh = plsc.VectorSubcoreMesh(
        core_axis_name="c", subcore_axis_name="s",
        num_cores=NC, num_subcores=NS)

    @functools.partial(
        pl.kernel,
        out_type=(
            jax.ShapeDtypeStruct((n_rows, dim), jnp.float32),
            jax.ShapeDtypeStruct((n_rows, dim), jnp.float32),
        ),
        mesh=mesh,
        scratch_types=[
            pltpu.VMEM((rows_per_tile,), jnp.int32),
            pltpu.VMEM((2, CHUNK, dim), jnp.float32),
            pltpu.VMEM((2, CHUNK, dim), jnp.float32),
            pltpu.VMEM((2, CHUNK, dim), jnp.float32),
            pltpu.VMEM((2, CHUNK, dim), jnp.float32),
            pltpu.VMEM((dim,), jnp.float32),
            pltpu.VMEM((dim,), jnp.float32),
            pltpu.SemaphoreType.DMA,
            pltpu.SemaphoreType.DMA,
            pltpu.SemaphoreType.DMA,
            pltpu.SemaphoreType.DMA,
            pltpu.SemaphoreType.DMA,
        ],
    )
    def sc_kernel(x_hbm, er_hbm, ei_hbm, g_hbm, b_hbm, outr_hbm, outi_hbm,
                  idx_v, re_v, im_v, or_v, oi_v, g_v, b_v,
                  sem_r0, sem_r1, sem_i0, sem_i1, sem_o):
        wid = lax.axis_index("s") * NC + lax.axis_index("c")
        tile_base = wid * rows_per_tile

        pltpu.sync_copy(g_hbm, g_v)
        pltpu.sync_copy(b_hbm, b_v)
        pltpu.sync_copy(x_hbm.at[pl.ds(tile_base, rows_per_tile)], idx_v)
        gs = [g_v[pl.ds(L * j, L)] for j in range(nj)]
        bs = [b_v[pl.ds(L * j, L)] for j in range(nj)]

        def gather_in(k, slot, semr, semi):
            idx = idx_v.at[pl.ds(k * CHUNK, CHUNK)]
            cr = pltpu.async_copy(er_hbm.at[idx], re_v.at[slot], semr)
            ci = pltpu.async_copy(ei_hbm.at[idx], im_v.at[slot], semi)
            return cr, ci

        def _one_row(slot, r):
            mags, ys = [], []
            acc = jnp.zeros((L,), jnp.float32)
            acc2 = jnp.zeros((L,), jnp.float32)
            for j in range(nj):
                re = re_v[slot, r, pl.ds(L * j, L)]
                im = im_v[slot, r, pl.ds(L * j, L)]
                msq = re * re + im * im
                y = _rsqrt_nr(msq, 1)
                mag = msq * y
                acc = acc + mag
                acc2 = acc2 + msq
                mags.append(mag)
                ys.append(y)
            mu_v = _lane_sum(acc) * (1.0 / dim)
            ex2 = _lane_sum(acc2) * (1.0 / dim)
            var = ex2 - mu_v * mu_v
            return mags, ys, mu_v, var

        def _finish_row(slot, r, mags, ys, mu_v, var):
            rs = _rsqrt_nr(var + 1e-5, 2)
            for j in range(nj):
                re = re_v[slot, r, pl.ds(L * j, L)]
                im = im_v[slot, r, pl.ds(L * j, L)]
                normed = (mags[j] - mu_v) * (rs * gs[j]) + bs[j]
                s = normed * ys[j]
                or_v[slot, r, pl.ds(L * j, L)] = re * s
                oi_v[slot, r, pl.ds(L * j, L)] = im * s

        def compute_and_emit(k, slot, cr, ci):
            cr.wait()
            ci.wait()

            @plsc.parallel_loop(0, CHUNK, 1, unroll=2)
            def _(r):
                st = _one_row(slot, r)
                _finish_row(slot, r, *st)
            base = tile_base + k * CHUNK
            co_r = pltpu.async_copy(or_v.at[slot],
                                    outr_hbm.at[pl.ds(base, CHUNK)], sem_o)
            co_i = pltpu.async_copy(oi_v.at[slot],
                                    outi_hbm.at[pl.ds(base, CHUNK)], sem_o)
            return co_r, co_i

        c0 = gather_in(0, 0, sem_r0, sem_i0)

        def pair_body(p, _):
            k0 = 2 * p
            c1 = gather_in(k0 + 1, 1, sem_r1, sem_i1)
            w0 = compute_and_emit(k0, 0, *c0)

            @pl.when(p + 1 < n_pairs)
            def _():
                gather_in(k0 + 2, 0, sem_r0, sem_i0)

            w1 = compute_and_emit(k0 + 1, 1, *c1)
            w0[0].wait()
            w0[1].wait()
            w1[0].wait()
            w1[1].wait()
            return _

        lax.fori_loop(0, n_pairs, pair_body, None)

    return sc_kernel


def kernel(x, embed_real, embed_imag, gamma, beta):
    b, h = x.shape
    v, d = embed_real.shape
    n = b * h
    xf = x.reshape(n)
    sc = _make_sc_kernel(n, d)
    outr, outi = sc(xf, embed_real, embed_imag, gamma, beta)
    return lax.complex(outr.reshape(b, h, d), outi.reshape(b, h, d))


# speedup vs baseline: 1.0515x; 1.0213x over previous
import functools

import jax
import jax.numpy as jnp
from jax import lax
from jax.experimental import pallas as pl
from jax.experimental.pallas import tpu as pltpu
from jax.experimental.pallas import tpu_sc as plsc

NC = 2
NS = 16
L = 16
CHUNK = 80


def _lane_sum(x):
    lanes = lax.iota(jnp.int32, L)
    for sh in (1, 2, 4, 8):
        perm = lanes ^ sh
        x = x + x.at[perm].get(mode="promise_in_bounds")
    return x


def _rsqrt_nr(x, iters):
    i = lax.bitcast_convert_type(x, jnp.int32)
    i = 0x5F3759DF - (i >> 1)
    y = lax.bitcast_convert_type(i, jnp.float32)
    xh = x * 0.5
    for _ in range(iters):
        y = y * (1.5 - xh * y * y)
    return y


@functools.lru_cache(maxsize=None)
def _make_sc_kernel(n_rows, dim):
    assert dim == 128 and n_rows % (NC * NS * CHUNK * 2) == 0
    rows_per_tile = n_rows // (NC * NS)
    n_pairs = rows_per_tile // (2 * CHUNK)
    nj = dim // L

    mesh = plsc.VectorSubcoreMesh(
        core_axis_name="c", subcore_axis_name="s",
        num_cores=NC, num_subcores=NS)

    @functools.partial(
        pl.kernel,
        out_type=(
            jax.ShapeDtypeStruct((n_rows, dim), jnp.float32),
            jax.ShapeDtypeStruct((n_rows, dim), jnp.float32),
        ),
        mesh=mesh,
        scratch_types=[
            pltpu.VMEM((rows_per_tile,), jnp.int32),
            pltpu.VMEM((2, CHUNK, dim), jnp.float32),
            pltpu.VMEM((2, CHUNK, dim), jnp.float32),
            pltpu.VMEM((2, CHUNK, dim), jnp.float32),
            pltpu.VMEM((2, CHUNK, dim), jnp.float32),
            pltpu.VMEM((dim,), jnp.float32),
            pltpu.VMEM((dim,), jnp.float32),
            pltpu.SemaphoreType.DMA,
            pltpu.SemaphoreType.DMA,
            pltpu.SemaphoreType.DMA,
            pltpu.SemaphoreType.DMA,
            pltpu.SemaphoreType.DMA,
        ],
    )
    def sc_kernel(x_hbm, er_hbm, ei_hbm, g_hbm, b_hbm, outr_hbm, outi_hbm,
                  idx_v, re_v, im_v, or_v, oi_v, g_v, b_v,
                  sem_r0, sem_r1, sem_i0, sem_i1, sem_o):
        wid = lax.axis_index("s") * NC + lax.axis_index("c")
        tile_base = wid * rows_per_tile

        pltpu.sync_copy(g_hbm, g_v)
        pltpu.sync_copy(b_hbm, b_v)
        pltpu.sync_copy(x_hbm.at[pl.ds(tile_base, rows_per_tile)], idx_v)
        gs = [g_v[pl.ds(L * j, L)] for j in range(nj)]
        bs = [b_v[pl.ds(L * j, L)] for j in range(nj)]

        def gather_in(k, slot, semr, semi):
            idx = idx_v.at[pl.ds(k * CHUNK, CHUNK)]
            cr = pltpu.async_copy(er_hbm.at[idx], re_v.at[slot], semr)
            ci = pltpu.async_copy(ei_hbm.at[idx], im_v.at[slot], semi)
            return cr, ci

        def _one_row(slot, r):
            mags, ys = [], []
            acc = jnp.zeros((L,), jnp.float32)
            acc2 = jnp.zeros((L,), jnp.float32)
            for j in range(nj):
                re = re_v[slot, r, pl.ds(L * j, L)]
                im = im_v[slot, r, pl.ds(L * j, L)]
                msq = re * re + im * im
                y = _rsqrt_nr(msq, 1)
                mag = msq * y
                acc = acc + mag
                acc2 = acc2 + msq
                mags.append(mag)
                ys.append(y)
            mu_v = _lane_sum(acc) * (1.0 / dim)
            ex2 = _lane_sum(acc2) * (1.0 / dim)
            var = ex2 - mu_v * mu_v
            return mags, ys, mu_v, var

        def _finish_row(slot, r, mags, ys, mu_v, var):
            rs = _rsqrt_nr(var + 1e-5, 2)
            for j in range(nj):
                re = re_v[slot, r, pl.ds(L * j, L)]
                im = im_v[slot, r, pl.ds(L * j, L)]
                normed = (mags[j] - mu_v) * (rs * gs[j]) + bs[j]
                s = normed * ys[j]
                or_v[slot, r, pl.ds(L * j, L)] = re * s
                oi_v[slot, r, pl.ds(L * j, L)] = im * s

        def compute_and_emit(k, slot, cr, ci):
            cr.wait()
            ci.wait()

            @plsc.parallel_loop(0, CHUNK, 1, unroll=1)
            def _(r):
                st = _one_row(slot, r)
                _finish_row(slot, r, *st)
            base = tile_base + k * CHUNK
            co_r = pltpu.async_copy(or_v.at[slot],
                                    outr_hbm.at[pl.ds(base, CHUNK)], sem_o)
            co_i = pltpu.async_copy(oi_v.at[slot],
                                    outi_hbm.at[pl.ds(base, CHUNK)], sem_o)
            return co_r, co_i

        c0 = gather_in(0, 0, sem_r0, sem_i0)

        def pair_body(p, _):
            k0 = 2 * p
            c1 = gather_in(k0 + 1, 1, sem_r1, sem_i1)
            w0 = compute_and_emit(k0, 0, *c0)

            @pl.when(p + 1 < n_pairs)
            def _():
                gather_in(k0 + 2, 0, sem_r0, sem_i0)

            w1 = compute_and_emit(k0 + 1, 1, *c1)
            w0[0].wait()
            w0[1].wait()
            w1[0].wait()
            w1[1].wait()
            return _

        lax.fori_loop(0, n_pairs, pair_body, None)

    return sc_kernel


def kernel(x, embed_real, embed_imag, gamma, beta):
    b, h = x.shape
    v, d = embed_real.shape
    n = b * h
    xf = x.reshape(n)
    sc = _make_sc_kernel(n, d)
    outr, outi = sc(xf, embed_real, embed_imag, gamma, beta)
    return lax.complex(outr.reshape(b, h, d), outi.reshape(b, h, d))
